# Initial kernel scaffold; baseline (speedup 1.0000x reference)
import jax
import jax.numpy as jnp
from jax.experimental import pallas as pl


def kernel(x, edge_index, batch, emb_chrg, emb_pdg, emb_pv, W_cont, b_cont, W_cat, b_cat, W_enc, b_enc, bn_g, bn_b, gcn_W1, gcn_b1, gcn_W2, gcn_b2, W_o1, b_o1, W_o2, b_o2, dW1, db1, dg1, dbt1, dW2, db2, dg2, dbt2, dW3, db3, dg3, dbt3, dW4, db4):
    raise NotImplementedError("write your pallas kernel here")


# speedup vs baseline: 16.2317x; 16.2317x over previous
import functools

import jax
import jax.numpy as jnp
from jax import lax
from jax.experimental import pallas as pl
from jax.experimental.pallas import tpu as pltpu
from jax.experimental.pallas import tpu_sc as plsc

N = 100000
E = 1600000
G = 1024
H = 16
BN = 2000
NBLK = N // BN

E_PAD = 1638400
EROWS = E_PAD // 128
ACC_ROWS = 100096
NPOOL = 102400
POOL_ACC = 1088

_mesh = plsc.VectorSubcoreMesh(
    core_axis_name="c", subcore_axis_name="s", num_cores=2, num_subcores=16)


def _elu(x):
    return jnp.where(x > 0, x, jnp.exp(jnp.minimum(x, 0.0)) - 1.0)


def _zero_rows(rows, n):
    def body(i, _):
        rows[i] = jnp.zeros((16,), jnp.float32)
        return 0
    lax.fori_loop(0, n, body, 0)


def _fill_rows(rows, n, val):
    def body(i, _):
        rows[i] = jnp.full((16,), val, jnp.float32)
        return 0
    lax.fori_loop(0, n, body, 0)


def _deg_body(dst_hbm, out_hbm, rows, dbuf, acc):
    cid = lax.axis_index("c")
    sid = lax.axis_index("s")
    _zero_rows(rows, 1024)
    base = sid * 6256
    for j in range(6):
        pltpu.sync_copy(rows, acc.at[pl.ds(base + j * 1024, 1024)])
    pltpu.sync_copy(rows.at[pl.ds(0, 112)], acc.at[pl.ds(base + 6144, 112)])
    _fill_rows(rows, 1024, 1.0)
    plsc.subcore_barrier()

    wid = sid * 2 + cid
    erow0 = wid * (EROWS // 32)

    def chunk(ci, _):
        r0 = erow0 + ci * 8
        pltpu.sync_copy(dst_hbm.at[pl.ds(r0, 8)], dbuf)
        for j in range(8):
            pltpu.sync_copy(rows.at[pl.ds(j * 128, 128)],
                            acc.at[dbuf.at[j]], add=True)
        return 0

    lax.fori_loop(0, (EROWS // 32) // 8, chunk, 0)
    plsc.subcore_barrier()
    _readout(acc, out_hbm, cid, sid)


def _readout(acc, out_hbm, cid, sid):
    ro = sid * 6240
    pltpu.sync_copy(acc.at[pl.ds(ro, 6240)], out_hbm.at[cid, pl.ds(ro, 6240)])

    @pl.when(sid == 0)
    def _():
        pltpu.sync_copy(acc.at[pl.ds(99840, 160)],
                        out_hbm.at[cid, pl.ds(99840, 160)])


_deg_call = pl.kernel(
    _deg_body,
    out_type=jax.ShapeDtypeStruct((2, N, H), jnp.float32),
    mesh=_mesh,
    compiler_params=pltpu.CompilerParams(use_tc_tiling_on_sc=False),
    scratch_types=[
        pltpu.VMEM((1024, 16), jnp.float32),
        pltpu.VMEM((8, 128), jnp.int32),
        pltpu.VMEM_SHARED((ACC_ROWS, 16), jnp.float32),
    ],
)


def _scat_body(h_hbm, src_hbm, dst_hbm, out_hbm, rows, sbuf, dbuf, acc, sem):
    cid = lax.axis_index("c")
    sid = lax.axis_index("s")
    _zero_rows(rows, 1024)
    base = sid * 6256
    for j in range(6):
        pltpu.sync_copy(rows, acc.at[pl.ds(base + j * 1024, 1024)])
    pltpu.sync_copy(rows.at[pl.ds(0, 112)], acc.at[pl.ds(base + 6144, 112)])
    plsc.subcore_barrier()

    erow0 = sid * (EROWS // 16)

    def chunk(ci, _):
        r0 = erow0 + ci * 8
        pltpu.sync_copy(src_hbm.at[cid, pl.ds(r0, 8)], sbuf)
        pltpu.sync_copy(dst_hbm.at[pl.ds(r0, 8)], dbuf)
        handles = [
            pltpu.async_copy(h_hbm.at[sbuf.at[j]],
                             rows.at[pl.ds(j * 128, 128)], sem)
            for j in range(8)
        ]
        for h in handles:
            h.wait()
        for j in range(8):
            pltpu.sync_copy(rows.at[pl.ds(j * 128, 128)],
                            acc.at[dbuf.at[j]], add=True)
        return 0

    lax.fori_loop(0, (EROWS // 16) // 8, chunk, 0)
    plsc.subcore_barrier()
    _readout(acc, out_hbm, cid, sid)


_scat_call = pl.kernel(
    _scat_body,
    out_type=jax.ShapeDtypeStruct((2, N, H), jnp.float32),
    mesh=_mesh,
    compiler_params=pltpu.CompilerParams(use_tc_tiling_on_sc=False),
    scratch_types=[
        pltpu.VMEM((1024, 16), jnp.float32),
        pltpu.VMEM((8, 128), jnp.int32),
        pltpu.VMEM((8, 128), jnp.int32),
        pltpu.VMEM_SHARED((ACC_ROWS, 16), jnp.float32),
        pltpu.SemaphoreType.DMA,
    ],
)


def _pool_body(p_hbm, b_hbm, out_hbm, rows, bbuf, acc):
    cid = lax.axis_index("c")
    sid = lax.axis_index("s")
    _zero_rows(rows, 68)
    pltpu.sync_copy(rows.at[pl.ds(0, 68)], acc.at[pl.ds(sid * 68, 68)])
    plsc.subcore_barrier()

    base = sid * (NPOOL // 16)
    brow0 = sid * (NPOOL // 16 // 128)
    for ci in range(3):
        pltpu.sync_copy(p_hbm.at[cid, pl.ds(base + ci * 2048, 2048)], rows)
        pltpu.sync_copy(b_hbm.at[pl.ds(brow0 + ci * 16, 16)], bbuf)
        for j in range(16):
            pltpu.sync_copy(rows.at[pl.ds(j * 128, 128)],
                            acc.at[bbuf.at[j]], add=True)
    pltpu.sync_copy(p_hbm.at[cid, pl.ds(base + 6144, 256)],
                    rows.at[pl.ds(0, 256)])
    pltpu.sync_copy(b_hbm.at[pl.ds(brow0 + 48, 2)], bbuf.at[pl.ds(0, 2)])
    for j in range(2):
        pltpu.sync_copy(rows.at[pl.ds(j * 128, 128)],
                        acc.at[bbuf.at[j]], add=True)
    plsc.subcore_barrier()
    pltpu.sync_copy(acc.at[pl.ds(sid * 64, 64)],
                    out_hbm.at[cid, pl.ds(sid * 64, 64)])


_pool_call = pl.kernel(
    _pool_body,
    out_type=jax.ShapeDtypeStruct((2, G, H), jnp.float32),
    mesh=_mesh,
    compiler_params=pltpu.CompilerParams(use_tc_tiling_on_sc=False),
    scratch_types=[
        pltpu.VMEM((2048, 16), jnp.float32),
        pltpu.VMEM((16, 128), jnp.int32),
        pltpu.VMEM_SHARED((POOL_ACC, 16), jnp.float32),
    ],
)


def _feat_body(x_ref, ech_ref, epdg_ref, epv_ref, wco_ref, bco_ref,
               wca_ref, bca_ref, wen_ref, ben_ref, z_ref, st_ref):
    xb = x_ref[...]
    xcat = xb[:, 8:11].astype(jnp.int32)
    e_cont = _elu(jnp.dot(xb[:, :7], wco_ref[...],
                          preferred_element_type=jnp.float32) + bco_ref[...])

    i_ch = jnp.clip(xcat[:, 1:2] + 1, 0, 2)
    oh = (i_ch == lax.broadcasted_iota(jnp.int32, (BN, 3), 1))
    e_ch = jnp.dot(oh.astype(jnp.float32), ech_ref[...],
                   preferred_element_type=jnp.float32)

    pdg = jnp.abs(xcat[:, 0:1])
    for i, pv in enumerate([1, 2, 11, 13, 22, 130, 211]):
        pdg = jnp.where(pdg == pv, jnp.full_like(pdg, i), pdg)
    pdg = jnp.clip(pdg, 0, 6)
    oh = (pdg == lax.broadcasted_iota(jnp.int32, (BN, 7), 1))
    e_pdg = jnp.dot(oh.astype(jnp.float32), epdg_ref[...],
                    preferred_element_type=jnp.float32)

    i_pv = jnp.clip(xcat[:, 2:3], 0, 3)
    oh = (i_pv == lax.broadcasted_iota(jnp.int32, (BN, 4), 1))
    e_pv = jnp.dot(oh.astype(jnp.float32), epv_ref[...],
                   preferred_element_type=jnp.float32)

    wca = wca_ref[...]
    e_cat = _elu(jnp.dot(e_ch, wca[0:8], preferred_element_type=jnp.float32)
                 + jnp.dot(e_pdg, wca[8:16], preferred_element_type=jnp.float32)
                 + jnp.dot(e_pv, wca[16:24], preferred_element_type=jnp.float32)
                 + bca_ref[...])
    wen = wen_ref[...]
    z = _elu(jnp.dot(e_cat, wen[0:16], preferred_element_type=jnp.float32)
             + jnp.dot(e_cont, wen[16:32], preferred_element_type=jnp.float32)
             + ben_ref[...])
    z_ref[...] = z

    s1 = jnp.sum(z, axis=0, keepdims=True)
    s2 = jnp.sum(z * z, axis=0, keepdims=True)
    upd = jnp.concatenate([s1, s2, jnp.zeros((6, 32), jnp.float32)], axis=0)

    @pl.when(pl.program_id(0) == 0)
    def _():
        st_ref[...] = jnp.zeros((8, 32), jnp.float32)

    st_ref[...] += upd


def _feat_call(x, emb_chrg, emb_pdg, emb_pv, W_cont, b_cont, W_cat, b_cat,
               W_enc, b_enc):
    full = lambda shape: pl.BlockSpec(shape, lambda i: (0, 0))
    return pl.pallas_call(
        _feat_body,
        grid=(NBLK,),
        in_specs=[
            pl.BlockSpec((BN, 11), lambda i: (i, 0)),
            full((3, 8)), full((7, 8)), full((4, 8)),
            full((7, 16)), full((1, 16)),
            full((24, 16)), full((1, 16)),
            full((32, 32)), full((1, 32)),
        ],
        out_specs=[
            pl.BlockSpec((BN, 32), lambda i: (i, 0)),
            pl.BlockSpec((8, 32), lambda i: (0, 0)),
        ],
        out_shape=[
            jax.ShapeDtypeStruct((N, 32), jnp.float32),
            jax.ShapeDtypeStruct((8, 32), jnp.float32),
        ],
    )(x, emb_chrg, emb_pdg, emb_pv, W_cont, b_cont, W_cat, b_cat,
      W_enc, b_enc)


def _pre1_body(z_ref, ac_ref, w1_ref, dga_ref, dgb_ref, h1_ref, dinv_ref):
    emb = z_ref[...] * ac_ref[0:1, :] + ac_ref[1:2, :]
    h = jnp.dot(emb, w1_ref[...], preferred_element_type=jnp.float32)
    deg = dga_ref[:, 0:1] + dgb_ref[:, 0:1] + 1.0
    dinv = lax.rsqrt(deg)
    hp = h * dinv
    h1_ref[0, :, :] = hp[:, :16]
    h1_ref[1, :, :] = hp[:, 16:]
    dinv_ref[...] = dinv


def _pre1_call(z, ac, W1, degA, degB):
    return pl.pallas_call(
        _pre1_body,
        grid=(NBLK,),
        in_specs=[
            pl.BlockSpec((BN, 32), lambda i: (i, 0)),
            pl.BlockSpec((8, 32), lambda i: (0, 0)),
            pl.BlockSpec((32, 32), lambda i: (0, 0)),
            pl.BlockSpec((BN, H), lambda i: (i, 0)),
            pl.BlockSpec((BN, H), lambda i: (i, 0)),
        ],
        out_specs=[
            pl.BlockSpec((2, BN, H), lambda i: (0, i, 0)),
            pl.BlockSpec((BN, 1), lambda i: (i, 0)),
        ],
        out_shape=[
            jax.ShapeDtypeStruct((2, N, H), jnp.float32),
            jax.ShapeDtypeStruct((N, 1), jnp.float32),
        ],
    )(z, ac, W1, degA, degB)


def _pre2_body(ya_ref, yb_ref, ha_ref, hb_ref, dinv_ref, b1_ref, w2_ref,
               h2_ref):
    dinv = dinv_ref[...]
    b1 = b1_ref[...]
    lo = jnp.maximum(dinv * (ya_ref[...] + ha_ref[...]) + b1[:, :16], 0.0)
    hi = jnp.maximum(dinv * (yb_ref[...] + hb_ref[...]) + b1[:, 16:], 0.0)
    w2 = w2_ref[...]
    h = (jnp.dot(lo, w2[0:16], preferred_element_type=jnp.float32)
         + jnp.dot(hi, w2[16:32], preferred_element_type=jnp.float32))
    hp = h * dinv
    h2_ref[0, :, :] = hp[:, :16]
    h2_ref[1, :, :] = hp[:, 16:]


def _pre2_call(ya, yb, ha, hb, dinv, b1, W2):
    return pl.pallas_call(
        _pre2_body,
        grid=(NBLK,),
        in_specs=[
            pl.BlockSpec((BN, H), lambda i: (i, 0)),
            pl.BlockSpec((BN, H), lambda i: (i, 0)),
            pl.BlockSpec((BN, H), lambda i: (i, 0)),
            pl.BlockSpec((BN, H), lambda i: (i, 0)),
            pl.BlockSpec((BN, 1), lambda i: (i, 0)),
            pl.BlockSpec((1, 32), lambda i: (0, 0)),
            pl.BlockSpec((32, 32), lambda i: (0, 0)),
        ],
        out_specs=pl.BlockSpec((2, BN, H), lambda i: (0, i, 0)),
        out_shape=jax.ShapeDtypeStruct((2, N, H), jnp.float32),
    )(ya, yb, ha, hb, dinv, b1, W2)


def _head_body(ya_ref, yb_ref, ha_ref, hb_ref, dinv_ref, b2_ref,
               wo1_ref, bo1_ref, wo2_ref, bo2_ref, out_ref, p_ref):
    dinv = dinv_ref[...]
    b2 = b2_ref[...]
    lo = jnp.maximum(dinv * (ya_ref[...] + ha_ref[...]) + b2[:, :16], 0.0)
    hi = jnp.maximum(dinv * (yb_ref[...] + hb_ref[...]) + b2[:, 16:], 0.0)
    wo1 = wo1_ref[...]
    t = _elu(jnp.dot(lo, wo1[0:16], preferred_element_type=jnp.float32)
             + jnp.dot(hi, wo1[16:32], preferred_element_type=jnp.float32)
             + bo1_ref[...])
    o = jnp.dot(t, wo2_ref[...], preferred_element_type=jnp.float32) \
        + bo2_ref[...]
    out_ref[...] = o
    p_ref[0, :, :] = lo * o
    p_ref[1, :, :] = hi * o


def _head_call(ya, yb, ha, hb, dinv, b2, W_o1, b_o1, W_o2, b_o2):
    return pl.pallas_call(
        _head_body,
        grid=(NBLK,),
        in_specs=[
            pl.BlockSpec((BN, H), lambda i: (i, 0)),
            pl.BlockSpec((BN, H), lambda i: (i, 0)),
            pl.BlockSpec((BN, H), lambda i: (i, 0)),
            pl.BlockSpec((BN, H), lambda i: (i, 0)),
            pl.BlockSpec((BN, 1), lambda i: (i, 0)),
            pl.BlockSpec((1, 32), lambda i: (0, 0)),
            pl.BlockSpec((32, 16), lambda i: (0, 0)),
            pl.BlockSpec((1, 16), lambda i: (0, 0)),
            pl.BlockSpec((16, 1), lambda i: (0, 0)),
            pl.BlockSpec((1, 1), lambda i: (0, 0)),
        ],
        out_specs=[
            pl.BlockSpec((BN, 1), lambda i: (i, 0)),
            pl.BlockSpec((2, BN, H), lambda i: (0, i, 0)),
        ],
        out_shape=[
            jax.ShapeDtypeStruct((N, 1), jnp.float32),
            jax.ShapeDtypeStruct((2, N, H), jnp.float32),
        ],
    )(ya, yb, ha, hb, dinv, b2, W_o1, b_o1, W_o2, b_o2)


def _bn_in(h, g, b):
    m = jnp.mean(h, axis=0, keepdims=True)
    v = jnp.mean((h - m) * (h - m), axis=0, keepdims=True)
    return g * (h - m) * lax.rsqrt(v + 1e-5) + b


def _mlp_body(pa_ref, pb_ref, w1_ref, b1_ref, g1_ref, t1_ref,
              w2_ref, b2_ref, g2_ref, t2_ref,
              w3_ref, b3_ref, g3_ref, t3_ref,
              w4_ref, b4_ref, out_ref):
    w1 = w1_ref[...]
    h = (jnp.dot(pa_ref[...], w1[0:16], preferred_element_type=jnp.float32)
         + jnp.dot(pb_ref[...], w1[16:32], preferred_element_type=jnp.float32)
         + b1_ref[...])
    h = _elu(_bn_in(h, g1_ref[...], t1_ref[...]))
    h = jnp.dot(h, w2_ref[...], preferred_element_type=jnp.float32) \
        + b2_ref[...]
    h = _elu(_bn_in(h, g2_ref[...], t2_ref[...]))
    h = jnp.dot(h, w3_ref[...], preferred_element_type=jnp.float32) \
        + b3_ref[...]
    h = _elu(_bn_in(h, g3_ref[...], t3_ref[...]))
    h = jnp.dot(h, w4_ref[...], preferred_element_type=jnp.float32) \
        + b4_ref[...]
    out_ref[...] = jnp.maximum(h, 0.0) + jnp.log1p(jnp.exp(-jnp.abs(h)))


def _mlp_call(pa, pb, dW1, db1, dg1, dbt1, dW2, db2, dg2, dbt2,
              dW3, db3, dg3, dbt3, dW4, db4):
    full = lambda shape: pl.BlockSpec(shape, lambda: (0, 0))
    return pl.pallas_call(
        _mlp_body,
        in_specs=[
            full((G, H)), full((G, H)),
            full((32, 32)), full((1, 32)), full((1, 32)), full((1, 32)),
            full((32, 32)), full((1, 32)), full((1, 32)), full((1, 32)),
            full((32, 32)), full((1, 32)), full((1, 32)), full((1, 32)),
            full((32, 5)), full((1, 5)),
        ],
        out_specs=full((G, 5)),
        out_shape=jax.ShapeDtypeStruct((G, 5), jnp.float32),
    )(pa, pb, dW1, db1, dg1, dbt1, dW2, db2, dg2, dbt2,
      dW3, db3, dg3, dbt3, dW4, db4)


def kernel(x, edge_index, batch, emb_chrg, emb_pdg, emb_pv, W_cont, b_cont,
           W_cat, b_cat, W_enc, b_enc, bn_g, bn_b, gcn_W1, gcn_b1, gcn_W2,
           gcn_b2, W_o1, b_o1, W_o2, b_o2, dW1, db1, dg1, dbt1, dW2, db2,
           dg2, dbt2, dW3, db3, dg3, dbt3, dW4, db4):
    src = edge_index[0].astype(jnp.int32)
    dst = edge_index[1].astype(jnp.int32)
    npad = E_PAD - E
    srcp = jnp.concatenate([src, jnp.zeros((npad,), jnp.int32)])
    dstp = jnp.concatenate(
        [dst, N + (jnp.arange(npad, dtype=jnp.int32) % (ACC_ROWS - N))])
    src_sh = jnp.stack([srcp, srcp + N]).reshape(2, EROWS, 128)
    dst2d = dstp.reshape(EROWS, 128)
    batchp = jnp.concatenate(
        [batch.astype(jnp.int32),
         jnp.full((NPOOL - N,), G, jnp.int32)]).reshape(NPOOL // 128, 128)

    z, stats = _feat_call(x, emb_chrg, emb_pdg, emb_pv, W_cont,
                          b_cont.reshape(1, 16), W_cat, b_cat.reshape(1, 16),
                          W_enc, b_enc.reshape(1, 32))
    m = stats[0] / N
    v = stats[1] / N - m * m
    a = bn_g * lax.rsqrt(v + 1e-5)
    c = bn_b - m * a
    ac = jnp.zeros((8, 32), jnp.float32).at[0].set(a).at[1].set(c)

    deg16 = _deg_call(dst2d)

    h1s, dinv = _pre1_call(z, ac, gcn_W1, deg16[0], deg16[1])
    y1 = _scat_call(h1s.reshape(2 * N, H), src_sh, dst2d)
    h2s = _pre2_call(y1[0], y1[1], h1s[0], h1s[1], dinv,
                     gcn_b1.reshape(1, 32), gcn_W2)
    y2 = _scat_call(h2s.reshape(2 * N, H), src_sh, dst2d)
    outv, ps = _head_call(y2[0], y2[1], h2s[0], h2s[1], dinv,
                          gcn_b2.reshape(1, 32), W_o1, b_o1.reshape(1, 16),
                          W_o2, b_o2.reshape(1, 1))
    ps_pad = jnp.pad(ps, ((0, 0), (0, NPOOL - N), (0, 0)))
    pool = _pool_call(ps_pad, batchp)
    out5 = _mlp_call(pool[0], pool[1], dW1, db1.reshape(1, 32), dg1.reshape(1, 32),
                     dbt1.reshape(1, 32), dW2, db2.reshape(1, 32),
                     dg2.reshape(1, 32), dbt2.reshape(1, 32), dW3,
                     db3.reshape(1, 32), dg3.reshape(1, 32),
                     dbt3.reshape(1, 32), dW4, db4.reshape(1, 5))
    return (outv.reshape(N), out5.T)


# speedup vs baseline: 17.7277x; 1.0922x over previous
import functools

import jax
import jax.numpy as jnp
from jax import lax
from jax.experimental import pallas as pl
from jax.experimental.pallas import tpu as pltpu
from jax.experimental.pallas import tpu_sc as plsc

N = 100000
E = 1600000
G = 1024
H = 16
BN = 2000
NBLK = N // BN

E_PAD = 1638400
EROWS = E_PAD // 128
ACC_ROWS = 100096
NPOOL = 102400
POOL_ACC = 1088

_mesh = plsc.VectorSubcoreMesh(
    core_axis_name="c", subcore_axis_name="s", num_cores=2, num_subcores=16)


def _elu(x):
    return jnp.where(x > 0, x, jnp.exp(jnp.minimum(x, 0.0)) - 1.0)


def _zero_rows(rows, n):
    def body(i, _):
        rows[i] = jnp.zeros((16,), jnp.float32)
        return 0
    lax.fori_loop(0, n, body, 0)


def _fill_rows(rows, n, val):
    def body(i, _):
        rows[i] = jnp.full((16,), val, jnp.float32)
        return 0
    lax.fori_loop(0, n, body, 0)


def _deg_body(dst_hbm, out_hbm, rows, dbuf, acc):
    cid = lax.axis_index("c")
    sid = lax.axis_index("s")
    _zero_rows(rows, 1024)
    base = sid * 6256
    for j in range(6):
        pltpu.sync_copy(rows, acc.at[pl.ds(base + j * 1024, 1024)])
    pltpu.sync_copy(rows.at[pl.ds(0, 112)], acc.at[pl.ds(base + 6144, 112)])
    _fill_rows(rows, 1024, 1.0)
    plsc.subcore_barrier()

    wid = sid * 2 + cid
    erow0 = wid * (EROWS // 32)

    def chunk(ci, _):
        r0 = erow0 + ci * 8
        pltpu.sync_copy(dst_hbm.at[pl.ds(r0, 8)], dbuf)
        for j in range(8):
            pltpu.sync_copy(rows.at[pl.ds(j * 128, 128)],
                            acc.at[dbuf.at[j]], add=True)
        return 0

    lax.fori_loop(0, (EROWS // 32) // 8, chunk, 0)
    plsc.subcore_barrier()
    _readout(acc, out_hbm, cid, sid)


def _readout(acc, out_hbm, cid, sid):
    ro = sid * 6240
    pltpu.sync_copy(acc.at[pl.ds(ro, 6240)], out_hbm.at[cid, pl.ds(ro, 6240)])

    @pl.when(sid == 0)
    def _():
        pltpu.sync_copy(acc.at[pl.ds(99840, 160)],
                        out_hbm.at[cid, pl.ds(99840, 160)])


_deg_call = pl.kernel(
    _deg_body,
    out_type=jax.ShapeDtypeStruct((2, N, H), jnp.float32),
    mesh=_mesh,
    compiler_params=pltpu.CompilerParams(use_tc_tiling_on_sc=False),
    scratch_types=[
        pltpu.VMEM((1024, 16), jnp.float32),
        pltpu.VMEM((8, 128), jnp.int32),
        pltpu.VMEM_SHARED((ACC_ROWS, 16), jnp.float32),
    ],
)


def _scat_body(h_hbm, src_hbm, dst_hbm, out_hbm, rows0, rows1, isb, idb,
               acc, gsem, ssem):
    cid = lax.axis_index("c")
    sid = lax.axis_index("s")
    _zero_rows(rows0, 512)
    base = sid * 6256
    for j in range(12):
        pltpu.sync_copy(rows0, acc.at[pl.ds(base + j * 512, 512)])
    pltpu.sync_copy(rows0.at[pl.ds(0, 112)], acc.at[pl.ds(base + 6144, 112)])
    plsc.subcore_barrier()

    erow0 = sid * (EROWS // 16)
    rows = (rows0, rows1)

    def fire_g(k, rb):
        return [pltpu.async_copy(h_hbm.at[isb.at[4 * k + j]],
                                 rb.at[pl.ds(j * 128, 128)], gsem)
                for j in range(4)]

    def block(bi, _):
        r0 = erow0 + bi * 32
        pltpu.sync_copy(src_hbm.at[cid, pl.ds(r0, 32)], isb)
        pltpu.sync_copy(dst_hbm.at[pl.ds(r0, 32)], idb)
        gh = fire_g(0, rows0)
        for k in range(8):
            rb = rows[k % 2]
            for h in gh:
                h.wait()
            if k < 7:
                gh = fire_g(k + 1, rows[(k + 1) % 2])
            sh = [pltpu.async_copy(rb.at[pl.ds(j * 128, 128)],
                                   acc.at[idb.at[4 * k + j]], ssem, add=True)
                  for j in range(4)]
            for h in sh:
                h.wait()
        return 0

    lax.fori_loop(0, (EROWS // 16) // 32, block, 0)
    plsc.subcore_barrier()
    _readout(acc, out_hbm, cid, sid)


_scat_call = pl.kernel(
    _scat_body,
    out_type=jax.ShapeDtypeStruct((2, N, H), jnp.float32),
    mesh=_mesh,
    compiler_params=pltpu.CompilerParams(use_tc_tiling_on_sc=False),
    scratch_types=[
        pltpu.VMEM((512, 16), jnp.float32),
        pltpu.VMEM((512, 16), jnp.float32),
        pltpu.VMEM((32, 128), jnp.int32),
        pltpu.VMEM((32, 128), jnp.int32),
        pltpu.VMEM_SHARED((ACC_ROWS, 16), jnp.float32),
        pltpu.SemaphoreType.DMA,
        pltpu.SemaphoreType.DMA,
    ],
)


def _pool_body(p_hbm, b_hbm, out_hbm, rows, bbuf, acc):
    cid = lax.axis_index("c")
    sid = lax.axis_index("s")
    _zero_rows(rows, 68)
    pltpu.sync_copy(rows.at[pl.ds(0, 68)], acc.at[pl.ds(sid * 68, 68)])
    plsc.subcore_barrier()

    base = sid * (NPOOL // 16)
    brow0 = sid * (NPOOL // 16 // 128)
    for ci in range(3):
        pltpu.sync_copy(p_hbm.at[cid, pl.ds(base + ci * 2048, 2048)], rows)
        pltpu.sync_copy(b_hbm.at[pl.ds(brow0 + ci * 16, 16)], bbuf)
        for j in range(16):
            pltpu.sync_copy(rows.at[pl.ds(j * 128, 128)],
                            acc.at[bbuf.at[j]], add=True)
    pltpu.sync_copy(p_hbm.at[cid, pl.ds(base + 6144, 256)],
                    rows.at[pl.ds(0, 256)])
    pltpu.sync_copy(b_hbm.at[pl.ds(brow0 + 48, 2)], bbuf.at[pl.ds(0, 2)])
    for j in range(2):
        pltpu.sync_copy(rows.at[pl.ds(j * 128, 128)],
                        acc.at[bbuf.at[j]], add=True)
    plsc.subcore_barrier()
    pltpu.sync_copy(acc.at[pl.ds(sid * 64, 64)],
                    out_hbm.at[cid, pl.ds(sid * 64, 64)])


_pool_call = pl.kernel(
    _pool_body,
    out_type=jax.ShapeDtypeStruct((2, G, H), jnp.float32),
    mesh=_mesh,
    compiler_params=pltpu.CompilerParams(use_tc_tiling_on_sc=False),
    scratch_types=[
        pltpu.VMEM((2048, 16), jnp.float32),
        pltpu.VMEM((16, 128), jnp.int32),
        pltpu.VMEM_SHARED((POOL_ACC, 16), jnp.float32),
    ],
)


def _feat_body(x_ref, ech_ref, epdg_ref, epv_ref, wco_ref, bco_ref,
               wca_ref, bca_ref, wen_ref, ben_ref, z_ref, st_ref):
    xb = x_ref[...]
    xcat = xb[:, 8:11].astype(jnp.int32)
    e_cont = _elu(jnp.dot(xb[:, :7], wco_ref[...],
                          preferred_element_type=jnp.float32) + bco_ref[...])

    i_ch = jnp.clip(xcat[:, 1:2] + 1, 0, 2)
    oh = (i_ch == lax.broadcasted_iota(jnp.int32, (BN, 3), 1))
    e_ch = jnp.dot(oh.astype(jnp.float32), ech_ref[...],
                   preferred_element_type=jnp.float32)

    pdg = jnp.abs(xcat[:, 0:1])
    for i, pv in enumerate([1, 2, 11, 13, 22, 130, 211]):
        pdg = jnp.where(pdg == pv, jnp.full_like(pdg, i), pdg)
    pdg = jnp.clip(pdg, 0, 6)
    oh = (pdg == lax.broadcasted_iota(jnp.int32, (BN, 7), 1))
    e_pdg = jnp.dot(oh.astype(jnp.float32), epdg_ref[...],
                    preferred_element_type=jnp.float32)

    i_pv = jnp.clip(xcat[:, 2:3], 0, 3)
    oh = (i_pv == lax.broadcasted_iota(jnp.int32, (BN, 4), 1))
    e_pv = jnp.dot(oh.astype(jnp.float32), epv_ref[...],
                   preferred_element_type=jnp.float32)

    wca = wca_ref[...]
    e_cat = _elu(jnp.dot(e_ch, wca[0:8], preferred_element_type=jnp.float32)
                 + jnp.dot(e_pdg, wca[8:16], preferred_element_type=jnp.float32)
                 + jnp.dot(e_pv, wca[16:24], preferred_element_type=jnp.float32)
                 + bca_ref[...])
    wen = wen_ref[...]
    z = _elu(jnp.dot(e_cat, wen[0:16], preferred_element_type=jnp.float32)
             + jnp.dot(e_cont, wen[16:32], preferred_element_type=jnp.float32)
             + ben_ref[...])
    z_ref[...] = z

    s1 = jnp.sum(z, axis=0, keepdims=True)
    s2 = jnp.sum(z * z, axis=0, keepdims=True)
    upd = jnp.concatenate([s1, s2, jnp.zeros((6, 32), jnp.float32)], axis=0)

    @pl.when(pl.program_id(0) == 0)
    def _():
        st_ref[...] = jnp.zeros((8, 32), jnp.float32)

    st_ref[...] += upd


def _feat_call(x, emb_chrg, emb_pdg, emb_pv, W_cont, b_cont, W_cat, b_cat,
               W_enc, b_enc):
    full = lambda shape: pl.BlockSpec(shape, lambda i: (0, 0))
    return pl.pallas_call(
        _feat_body,
        grid=(NBLK,),
        in_specs=[
            pl.BlockSpec((BN, 11), lambda i: (i, 0)),
            full((3, 8)), full((7, 8)), full((4, 8)),
            full((7, 16)), full((1, 16)),
            full((24, 16)), full((1, 16)),
            full((32, 32)), full((1, 32)),
        ],
        out_specs=[
            pl.BlockSpec((BN, 32), lambda i: (i, 0)),
            pl.BlockSpec((8, 32), lambda i: (0, 0)),
        ],
        out_shape=[
            jax.ShapeDtypeStruct((N, 32), jnp.float32),
            jax.ShapeDtypeStruct((8, 32), jnp.float32),
        ],
    )(x, emb_chrg, emb_pdg, emb_pv, W_cont, b_cont, W_cat, b_cat,
      W_enc, b_enc)


def _pre1_body(z_ref, ac_ref, w1_ref, dga_ref, dgb_ref, h1_ref, dinv_ref):
    emb = z_ref[...] * ac_ref[0:1, :] + ac_ref[1:2, :]
    h = jnp.dot(emb, w1_ref[...], preferred_element_type=jnp.float32)
    deg = dga_ref[:, 0:1] + dgb_ref[:, 0:1] + 1.0
    dinv = lax.rsqrt(deg)
    hp = h * dinv
    h1_ref[0, :, :] = hp[:, :16]
    h1_ref[1, :, :] = hp[:, 16:]
    dinv_ref[...] = dinv


def _pre1_call(z, ac, W1, degA, degB):
    return pl.pallas_call(
        _pre1_body,
        grid=(NBLK,),
        in_specs=[
            pl.BlockSpec((BN, 32), lambda i: (i, 0)),
            pl.BlockSpec((8, 32), lambda i: (0, 0)),
            pl.BlockSpec((32, 32), lambda i: (0, 0)),
            pl.BlockSpec((BN, H), lambda i: (i, 0)),
            pl.BlockSpec((BN, H), lambda i: (i, 0)),
        ],
        out_specs=[
            pl.BlockSpec((2, BN, H), lambda i: (0, i, 0)),
            pl.BlockSpec((BN, 1), lambda i: (i, 0)),
        ],
        out_shape=[
            jax.ShapeDtypeStruct((2, N, H), jnp.float32),
            jax.ShapeDtypeStruct((N, 1), jnp.float32),
        ],
    )(z, ac, W1, degA, degB)


def _pre2_body(ya_ref, yb_ref, ha_ref, hb_ref, dinv_ref, b1_ref, w2_ref,
               h2_ref):
    dinv = dinv_ref[...]
    b1 = b1_ref[...]
    lo = jnp.maximum(dinv * (ya_ref[...] + ha_ref[...]) + b1[:, :16], 0.0)
    hi = jnp.maximum(dinv * (yb_ref[...] + hb_ref[...]) + b1[:, 16:], 0.0)
    w2 = w2_ref[...]
    h = (jnp.dot(lo, w2[0:16], preferred_element_type=jnp.float32)
         + jnp.dot(hi, w2[16:32], preferred_element_type=jnp.float32))
    hp = h * dinv
    h2_ref[0, :, :] = hp[:, :16]
    h2_ref[1, :, :] = hp[:, 16:]


def _pre2_call(ya, yb, ha, hb, dinv, b1, W2):
    return pl.pallas_call(
        _pre2_body,
        grid=(NBLK,),
        in_specs=[
            pl.BlockSpec((BN, H), lambda i: (i, 0)),
            pl.BlockSpec((BN, H), lambda i: (i, 0)),
            pl.BlockSpec((BN, H), lambda i: (i, 0)),
            pl.BlockSpec((BN, H), lambda i: (i, 0)),
            pl.BlockSpec((BN, 1), lambda i: (i, 0)),
            pl.BlockSpec((1, 32), lambda i: (0, 0)),
            pl.BlockSpec((32, 32), lambda i: (0, 0)),
        ],
        out_specs=pl.BlockSpec((2, BN, H), lambda i: (0, i, 0)),
        out_shape=jax.ShapeDtypeStruct((2, N, H), jnp.float32),
    )(ya, yb, ha, hb, dinv, b1, W2)


def _head_body(ya_ref, yb_ref, ha_ref, hb_ref, dinv_ref, b2_ref,
               wo1_ref, bo1_ref, wo2_ref, bo2_ref, out_ref, p_ref):
    dinv = dinv_ref[...]
    b2 = b2_ref[...]
    lo = jnp.maximum(dinv * (ya_ref[...] + ha_ref[...]) + b2[:, :16], 0.0)
    hi = jnp.maximum(dinv * (yb_ref[...] + hb_ref[...]) + b2[:, 16:], 0.0)
    wo1 = wo1_ref[...]
    t = _elu(jnp.dot(lo, wo1[0:16], preferred_element_type=jnp.float32)
             + jnp.dot(hi, wo1[16:32], preferred_element_type=jnp.float32)
             + bo1_ref[...])
    o = jnp.dot(t, wo2_ref[...], preferred_element_type=jnp.float32) \
        + bo2_ref[...]
    out_ref[...] = o
    p_ref[0, :, :] = lo * o
    p_ref[1, :, :] = hi * o


def _head_call(ya, yb, ha, hb, dinv, b2, W_o1, b_o1, W_o2, b_o2):
    return pl.pallas_call(
        _head_body,
        grid=(NBLK,),
        in_specs=[
            pl.BlockSpec((BN, H), lambda i: (i, 0)),
            pl.BlockSpec((BN, H), lambda i: (i, 0)),
            pl.BlockSpec((BN, H), lambda i: (i, 0)),
            pl.BlockSpec((BN, H), lambda i: (i, 0)),
            pl.BlockSpec((BN, 1), lambda i: (i, 0)),
            pl.BlockSpec((1, 32), lambda i: (0, 0)),
            pl.BlockSpec((32, 16), lambda i: (0, 0)),
            pl.BlockSpec((1, 16), lambda i: (0, 0)),
            pl.BlockSpec((16, 1), lambda i: (0, 0)),
            pl.BlockSpec((1, 1), lambda i: (0, 0)),
        ],
        out_specs=[
            pl.BlockSpec((BN, 1), lambda i: (i, 0)),
            pl.BlockSpec((2, BN, H), lambda i: (0, i, 0)),
        ],
        out_shape=[
            jax.ShapeDtypeStruct((N, 1), jnp.float32),
            jax.ShapeDtypeStruct((2, N, H), jnp.float32),
        ],
    )(ya, yb, ha, hb, dinv, b2, W_o1, b_o1, W_o2, b_o2)


def _bn_in(h, g, b):
    m = jnp.mean(h, axis=0, keepdims=True)
    v = jnp.mean((h - m) * (h - m), axis=0, keepdims=True)
    return g * (h - m) * lax.rsqrt(v + 1e-5) + b


def _mlp_body(pa_ref, pb_ref, w1_ref, b1_ref, g1_ref, t1_ref,
              w2_ref, b2_ref, g2_ref, t2_ref,
              w3_ref, b3_ref, g3_ref, t3_ref,
              w4_ref, b4_ref, out_ref):
    w1 = w1_ref[...]
    h = (jnp.dot(pa_ref[...], w1[0:16], preferred_element_type=jnp.float32)
         + jnp.dot(pb_ref[...], w1[16:32], preferred_element_type=jnp.float32)
         + b1_ref[...])
    h = _elu(_bn_in(h, g1_ref[...], t1_ref[...]))
    h = jnp.dot(h, w2_ref[...], preferred_element_type=jnp.float32) \
        + b2_ref[...]
    h = _elu(_bn_in(h, g2_ref[...], t2_ref[...]))
    h = jnp.dot(h, w3_ref[...], preferred_element_type=jnp.float32) \
        + b3_ref[...]
    h = _elu(_bn_in(h, g3_ref[...], t3_ref[...]))
    h = jnp.dot(h, w4_ref[...], preferred_element_type=jnp.float32) \
        + b4_ref[...]
    out_ref[...] = jnp.maximum(h, 0.0) + jnp.log1p(jnp.exp(-jnp.abs(h)))


def _mlp_call(pa, pb, dW1, db1, dg1, dbt1, dW2, db2, dg2, dbt2,
              dW3, db3, dg3, dbt3, dW4, db4):
    full = lambda shape: pl.BlockSpec(shape, lambda: (0, 0))
    return pl.pallas_call(
        _mlp_body,
        in_specs=[
            full((G, H)), full((G, H)),
            full((32, 32)), full((1, 32)), full((1, 32)), full((1, 32)),
            full((32, 32)), full((1, 32)), full((1, 32)), full((1, 32)),
            full((32, 32)), full((1, 32)), full((1, 32)), full((1, 32)),
            full((32, 5)), full((1, 5)),
        ],
        out_specs=full((G, 5)),
        out_shape=jax.ShapeDtypeStruct((G, 5), jnp.float32),
    )(pa, pb, dW1, db1, dg1, dbt1, dW2, db2, dg2, dbt2,
      dW3, db3, dg3, dbt3, dW4, db4)


def kernel(x, edge_index, batch, emb_chrg, emb_pdg, emb_pv, W_cont, b_cont,
           W_cat, b_cat, W_enc, b_enc, bn_g, bn_b, gcn_W1, gcn_b1, gcn_W2,
           gcn_b2, W_o1, b_o1, W_o2, b_o2, dW1, db1, dg1, dbt1, dW2, db2,
           dg2, dbt2, dW3, db3, dg3, dbt3, dW4, db4):
    src = edge_index[0].astype(jnp.int32)
    dst = edge_index[1].astype(jnp.int32)
    npad = E_PAD - E
    srcp = jnp.concatenate([src, jnp.zeros((npad,), jnp.int32)])
    dstp = jnp.concatenate(
        [dst, N + (jnp.arange(npad, dtype=jnp.int32) % (ACC_ROWS - N))])
    src_sh = jnp.stack([srcp, srcp + N]).reshape(2, EROWS, 128)
    dst2d = dstp.reshape(EROWS, 128)
    batchp = jnp.concatenate(
        [batch.astype(jnp.int32),
         jnp.full((NPOOL - N,), G, jnp.int32)]).reshape(NPOOL // 128, 128)

    z, stats = _feat_call(x, emb_chrg, emb_pdg, emb_pv, W_cont,
                          b_cont.reshape(1, 16), W_cat, b_cat.reshape(1, 16),
                          W_enc, b_enc.reshape(1, 32))
    m = stats[0] / N
    v = stats[1] / N - m * m
    a = bn_g * lax.rsqrt(v + 1e-5)
    c = bn_b - m * a
    ac = jnp.zeros((8, 32), jnp.float32).at[0].set(a).at[1].set(c)

    deg16 = _deg_call(dst2d)

    h1s, dinv = _pre1_call(z, ac, gcn_W1, deg16[0], deg16[1])
    y1 = _scat_call(h1s.reshape(2 * N, H), src_sh, dst2d)
    h2s = _pre2_call(y1[0], y1[1], h1s[0], h1s[1], dinv,
                     gcn_b1.reshape(1, 32), gcn_W2)
    y2 = _scat_call(h2s.reshape(2 * N, H), src_sh, dst2d)
    outv, ps = _head_call(y2[0], y2[1], h2s[0], h2s[1], dinv,
                          gcn_b2.reshape(1, 32), W_o1, b_o1.reshape(1, 16),
                          W_o2, b_o2.reshape(1, 1))
    ps_pad = jnp.pad(ps, ((0, 0), (0, NPOOL - N), (0, 0)))
    pool = _pool_call(ps_pad, batchp)
    out5 = _mlp_call(pool[0], pool[1], dW1, db1.reshape(1, 32), dg1.reshape(1, 32),
                     dbt1.reshape(1, 32), dW2, db2.reshape(1, 32),
                     dg2.reshape(1, 32), dbt2.reshape(1, 32), dW3,
                     db3.reshape(1, 32), dg3.reshape(1, 32),
                     dbt3.reshape(1, 32), dW4, db4.reshape(1, 5))
    return (outv.reshape(N), out5.T)
